# Initial kernel scaffold; baseline (speedup 1.0000x reference)
import jax
import jax.numpy as jnp
from jax.experimental import pallas as pl


def kernel(x, pos_table):
    raise NotImplementedError("write your pallas kernel here")


# speedup vs baseline: 3.2122x; 3.2122x over previous
import jax
import jax.numpy as jnp
from jax.experimental import pallas as pl

_BS = 256


def _add_body(x_ref, p_ref, o_ref):
    o_ref[...] = x_ref[...] + p_ref[...]


def kernel(x, pos_table):
    batch, seq_len, d_model = x.shape
    table = pos_table[:seq_len]
    grid = (seq_len // _BS,)
    return pl.pallas_call(
        _add_body,
        grid=grid,
        in_specs=[
            pl.BlockSpec((batch, _BS, d_model), lambda s: (0, s, 0)),
            pl.BlockSpec((_BS, d_model), lambda s: (s, 0)),
        ],
        out_specs=pl.BlockSpec((batch, _BS, d_model), lambda s: (0, s, 0)),
        out_shape=jax.ShapeDtypeStruct((batch, seq_len, d_model), x.dtype),
    )(x, table)


# speedup vs baseline: 3.2171x; 1.0015x over previous
import jax
import jax.numpy as jnp
from jax.experimental import pallas as pl

_BS = 512


def _add_body(x_ref, p_ref, o_ref):
    o_ref[...] = x_ref[...] + p_ref[...]


def kernel(x, pos_table):
    batch, seq_len, d_model = x.shape
    table = pos_table[:seq_len]
    grid = (seq_len // _BS,)
    return pl.pallas_call(
        _add_body,
        grid=grid,
        in_specs=[
            pl.BlockSpec((batch, _BS, d_model), lambda s: (0, s, 0)),
            pl.BlockSpec((_BS, d_model), lambda s: (s, 0)),
        ],
        out_specs=pl.BlockSpec((batch, _BS, d_model), lambda s: (0, s, 0)),
        out_shape=jax.ShapeDtypeStruct((batch, seq_len, d_model), x.dtype),
    )(x, table)
